# baseline (speedup 1.0000x reference)
import functools

import jax
import jax.numpy as jnp
from jax import lax
from jax.experimental import pallas as pl
from jax.experimental.pallas import tpu as pltpu
from jax.experimental.pallas import tpu_sc as plsc

B, C, D = 4, 4096, 1024
NC, NS = 2, 16
NW = NC * NS
C_PER_W = C // NW
RC = 16
NCHUNK = C_PER_W // RC
CH = RC * D
NVEC = CH // 16


def _sc_body(x_hbm, emb_hbm, out_hbm, ebuf, xbuf, sem):
    cid = lax.axis_index("c")
    sid = lax.axis_index("s")
    w = sid * NC + cid
    c0 = w * C_PER_W

    def chunk(k, carry):
        base = pl.multiple_of((c0 + k * RC) * D, CH)
        cp_e = pltpu.async_copy(emb_hbm.at[pl.ds(base, CH)], ebuf, sem)
        cps = []
        for b in range(B):
            cps.append(pltpu.async_copy(
                x_hbm.at[pl.ds(b * (C * D) + base, CH)], xbuf.at[b], sem))
        cp_e.wait()
        for cp in cps:
            cp.wait()

        def addv(j, c2):
            for u in range(8):
                off = (j * 8 + u) * 16
                e = ebuf[pl.ds(off, 16)]
                for b in range(B):
                    xv = xbuf[b, pl.ds(off, 16)]
                    xbuf[b, pl.ds(off, 16)] = xv + e
            return c2

        lax.fori_loop(0, NVEC // 8, addv, 0)

        for b in range(B):
            pltpu.sync_copy(xbuf.at[b], out_hbm.at[pl.ds(b * (C * D) + base, CH)])
        return carry

    lax.fori_loop(0, NCHUNK, chunk, 0)


@functools.partial(
    pl.kernel,
    mesh=plsc.VectorSubcoreMesh(core_axis_name="c", subcore_axis_name="s"),
    out_type=jax.ShapeDtypeStruct((B * C * D,), jnp.float32),
    scratch_types=[
        pltpu.VMEM((CH,), jnp.float32),
        pltpu.VMEM((B, CH), jnp.float32),
        pltpu.SemaphoreType.DMA,
    ],
)
def _sc_kernel(x_hbm, emb_hbm, out_hbm, ebuf, xbuf, sem):
    _sc_body(x_hbm, emb_hbm, out_hbm, ebuf, xbuf, sem)


def kernel(x, embedding):
    b, c, d = x.shape
    out = _sc_kernel(x.reshape(-1), embedding.reshape(-1))
    return out.reshape(b, c, d)


# speedup vs baseline: 1.2418x; 1.2418x over previous
import functools

import jax
import jax.numpy as jnp
from jax import lax
from jax.experimental import pallas as pl
from jax.experimental.pallas import tpu as pltpu
from jax.experimental.pallas import tpu_sc as plsc

B, C, D = 4, 4096, 1024
NC, NS = 2, 16
NW = NC * NS
C_PER_W = C // NW
RC = 8
NCHUNK = C_PER_W // RC
CH = RC * D
NVEC = CH // 16


def _sc_body(x_hbm, emb_hbm, out_hbm, ebuf, xbuf, sem_in, sem_out):
    cid = lax.axis_index("c")
    sid = lax.axis_index("s")
    w = sid * NC + cid
    c0 = w * C_PER_W

    def fire_loads(k, p):
        base = pl.multiple_of((c0 + k * RC) * D, CH)
        cps = [pltpu.async_copy(emb_hbm.at[pl.ds(base, CH)], ebuf.at[p], sem_in)]
        for b in range(B):
            cps.append(pltpu.async_copy(
                x_hbm.at[pl.ds(b * (C * D) + base, CH)], xbuf.at[p, b], sem_in))
        return cps

    def fire_stores(k, p):
        base = pl.multiple_of((c0 + k * RC) * D, CH)
        return [pltpu.async_copy(
            xbuf.at[p, b], out_hbm.at[pl.ds(b * (C * D) + base, CH)], sem_out)
            for b in range(B)]

    loads = fire_loads(0, 0)
    stores = []
    for k in range(NCHUNK):
        p = k % 2
        if k + 1 < NCHUNK:
            for cp in stores:
                cp.wait()
            next_loads = fire_loads(k + 1, (k + 1) % 2)
        else:
            next_loads = []
        for cp in loads:
            cp.wait()

        @functools.partial(plsc.parallel_loop, 0, NVEC, unroll=8)
        def _add(j):
            off = j * 16
            e = ebuf[p, pl.ds(off, 16)]
            for b in range(B):
                xv = xbuf[p, b, pl.ds(off, 16)]
                xbuf[p, b, pl.ds(off, 16)] = xv + e

        stores = fire_stores(k, p)
        loads = next_loads
    for cp in stores:
        cp.wait()


@functools.partial(
    pl.kernel,
    mesh=plsc.VectorSubcoreMesh(core_axis_name="c", subcore_axis_name="s"),
    out_type=jax.ShapeDtypeStruct((B * C * D,), jnp.float32),
    scratch_types=[
        pltpu.VMEM((2, CH), jnp.float32),
        pltpu.VMEM((2, B, CH), jnp.float32),
        pltpu.SemaphoreType.DMA,
        pltpu.SemaphoreType.DMA,
    ],
)
def _sc_kernel(x_hbm, emb_hbm, out_hbm, ebuf, xbuf, sem_in, sem_out):
    _sc_body(x_hbm, emb_hbm, out_hbm, ebuf, xbuf, sem_in, sem_out)


def kernel(x, embedding):
    b, c, d = x.shape
    out = _sc_kernel(x.reshape(-1), embedding.reshape(-1))
    return out.reshape(b, c, d)
